# baseline (speedup 1.0000x reference)
import functools

import jax
import jax.numpy as jnp
from jax import lax
from jax.experimental import pallas as pl
from jax.experimental.pallas import tpu as pltpu
from jax.experimental.pallas import tpu_sc as plsc

_SLOPE = 0.01
_EPS = 1e-5


def _leaky(x):
    return jnp.where(x > 0, x, _SLOPE * x)


def _fill_vmem(ref, rows, cols, val):
    v = jnp.full((16,), val, jnp.float32)

    def outer(r, _):
        def inner(cc, __):
            ref[r, pl.ds(cc * 16, 16)] = v
            return 0
        return lax.fori_loop(0, cols // 16, inner, 0)

    lax.fori_loop(0, rows, outer, 0)


def _sc_aggregate(y, src_r, dst_r, n_acc, n_out, with_cnt):
    F = y.shape[1]
    NCH = src_r.shape[1]
    NB = 2
    NJ = NCH // NB
    mesh = plsc.VectorSubcoreMesh(core_axis_name="c", subcore_axis_name="s")
    zr = n_acc // 16
    rpt = n_out // 16

    out_type = [jax.ShapeDtypeStruct((2, n_out, F), jnp.float32)]
    scratch = [
        pltpu.VMEM((NCH, 128), jnp.int32),
        pltpu.VMEM((NCH, 128), jnp.int32),
        pltpu.VMEM_SHARED((n_acc, F), jnp.float32),
    ]
    scratch += [pltpu.VMEM((128, F), jnp.float32) for _ in range(NB)]
    scratch += [pltpu.SemaphoreType.DMA for _ in range(NB)]
    scratch += [pltpu.SemaphoreType.DMA]
    if with_cnt:
        out_type.append(jax.ShapeDtypeStruct((32, n_acc), jnp.float32))
        scratch.append(pltpu.VMEM((n_acc,), jnp.float32))

    def body(y_hbm, src_hbm, dst_hbm, out_hbm, *rest):
        if with_cnt:
            cnt_hbm, rest = rest[0], rest[1:]
        src_v, dst_v, acc_sh = rest[:3]
        rows = rest[3:3 + NB]
        gsems = rest[3 + NB:3 + 2 * NB]
        ssem = rest[3 + 2 * NB]
        cnt_v = rest[4 + 2 * NB] if with_cnt else None
        cid = lax.axis_index("c")
        sid = lax.axis_index("s")
        wid = cid * 16 + sid
        ones16 = jnp.full((16,), 1.0, jnp.float32)
        zero16 = jnp.zeros((16,), jnp.float32)

        _fill_vmem(rows[0], 128, F, 0.0)

        def zacc(k, _):
            pltpu.sync_copy(rows[0], acc_sh.at[pl.ds(sid * zr + k * 128, 128)])
            return 0
        lax.fori_loop(0, zr // 128, zacc, 0)
        if with_cnt:
            def zcnt(i, _):
                cnt_v[pl.ds(i * 16, 16)] = zero16
                return 0
            lax.fori_loop(0, n_acc // 16, zcnt, 0)

        pltpu.sync_copy(src_hbm.at[wid], src_v)
        pltpu.sync_copy(dst_hbm.at[wid], dst_v)
        plsc.subcore_barrier()

        for b in range(NB):
            pltpu.async_copy(y_hbm.at[src_v.at[b]], rows[b], gsems[b])

        def step(jj, _):
            for b in range(NB):
                j = jj * NB + b
                pltpu.make_async_copy(y_hbm.at[src_v.at[j]], rows[b],
                                      gsems[b]).wait()
                d = pltpu.async_copy(rows[b], acc_sh.at[dst_v.at[j]], ssem,
                                     add=True)
                if with_cnt:
                    for c in range(8):
                        idx = dst_v[j, pl.ds(c * 16, 16)]
                        plsc.addupdate_scatter(cnt_v, [idx], ones16)
                d.wait()

                @pl.when(jj < NJ - 1)
                def _():
                    pltpu.async_copy(y_hbm.at[src_v.at[j + NB]], rows[b],
                                     gsems[b])
            return 0
        lax.fori_loop(0, NJ, step, 0)

        plsc.subcore_barrier()
        pltpu.sync_copy(acc_sh.at[pl.ds(sid * rpt, rpt)],
                        out_hbm.at[cid, pl.ds(sid * rpt, rpt)])
        if with_cnt:
            pltpu.sync_copy(cnt_v, cnt_hbm.at[wid])

    fn = pl.kernel(body, out_type=out_type, mesh=mesh, scratch_types=scratch,
                   compiler_params=pltpu.CompilerParams(
                       use_tc_tiling_on_sc=False, needs_layout_passes=False))
    return fn(y, src_r, dst_r)


def _prep_edges(ei, n_nodes, nch, n_acc):
    src, dst = ei[0], ei[1]
    e = src.shape[0]
    spare = n_acc - n_nodes
    dstp = jnp.where(src != dst, dst,
                     n_nodes + (jnp.arange(e, dtype=jnp.int32) % spare))
    ep = 32 * nch * 128
    pad = ep - e
    src = jnp.concatenate([src, jnp.zeros((pad,), jnp.int32)])
    dstp = jnp.concatenate(
        [dstp, n_nodes + (jnp.arange(pad, dtype=jnp.int32) % spare)])
    return src.reshape(32, nch, 128), dstp.reshape(32, nch, 128)


def _tc_matmul(x, w, bm):
    n, k = x.shape
    f = w.shape[1]

    def body(x_ref, w_ref, o_ref):
        o_ref[:] = jnp.dot(x_ref[:], w_ref[:], preferred_element_type=jnp.float32)

    return pl.pallas_call(
        body,
        grid=(n // bm,),
        in_specs=[pl.BlockSpec((bm, k), lambda i: (i, 0)),
                  pl.BlockSpec((k, f), lambda i: (0, 0))],
        out_specs=pl.BlockSpec((bm, f), lambda i: (i, 0)),
        out_shape=jax.ShapeDtypeStruct((n, f), jnp.float32),
    )(x, w)


def _tc_combine_stats(parts, cntp, y, b, bm):
    n, f = y.shape
    steps = n // bm

    def body(p_ref, c_ref, y_ref, b_ref, pre_ref, st_ref, sacc):
        i = pl.program_id(0)

        @pl.when(i == 0)
        def _():
            sacc[:] = jnp.zeros_like(sacc)

        cnt = jnp.sum(c_ref[:], axis=1, keepdims=True) + 1.0
        pre = (p_ref[0] + p_ref[1] + y_ref[:]) / cnt + b_ref[:]
        pre_ref[:] = pre
        sacc[0:1, :] += jnp.sum(pre, axis=0, keepdims=True)
        sacc[1:2, :] += jnp.sum(pre * pre, axis=0, keepdims=True)

        @pl.when(i == steps - 1)
        def _():
            st_ref[:] = sacc[:]

    return pl.pallas_call(
        body,
        grid=(steps,),
        in_specs=[pl.BlockSpec((2, bm, f), lambda i: (0, i, 0)),
                  pl.BlockSpec((bm, 32), lambda i: (i, 0)),
                  pl.BlockSpec((bm, f), lambda i: (i, 0)),
                  pl.BlockSpec((1, f), lambda i: (0, 0))],
        out_specs=[pl.BlockSpec((bm, f), lambda i: (i, 0)),
                   pl.BlockSpec((2, f), lambda i: (0, 0))],
        out_shape=[jax.ShapeDtypeStruct((n, f), jnp.float32),
                   jax.ShapeDtypeStruct((2, f), jnp.float32)],
        scratch_shapes=[pltpu.VMEM((2, f), jnp.float32)],
    )(parts, cntp, y, b)


def _bn_from_stats(st_ref, n):
    mean = st_ref[0:1, :] / n
    var = st_ref[1:2, :] / n - (st_ref[0:1, :] / n) ** 2
    inv = lax.rsqrt(var + _EPS)
    return mean, inv


def _tc_bn_split(pre, stats, gcat, bcat, l1w, fg, bm):
    n, f = pre.shape
    fl = f - fg
    fo = l1w.shape[1]

    def body(pre_ref, st_ref, g_ref, b_ref, w_ref, xg_ref, xl_ref, yl_ref):
        mean, inv = _bn_from_stats(st_ref, float(n))
        z = (pre_ref[:] - mean) * inv * g_ref[:] + b_ref[:]
        a = _leaky(z)
        xg_ref[:] = a[:, :fg]
        xl = a[:, fg:]
        xl_ref[:] = xl
        yl_ref[:] = jnp.dot(xl, w_ref[:], preferred_element_type=jnp.float32)

    return pl.pallas_call(
        body,
        grid=(n // bm,),
        in_specs=[pl.BlockSpec((bm, f), lambda i: (i, 0)),
                  pl.BlockSpec((2, f), lambda i: (0, 0)),
                  pl.BlockSpec((1, f), lambda i: (0, 0)),
                  pl.BlockSpec((1, f), lambda i: (0, 0)),
                  pl.BlockSpec((fl, fo), lambda i: (0, 0))],
        out_specs=[pl.BlockSpec((bm, fg), lambda i: (i, 0)),
                   pl.BlockSpec((bm, fl), lambda i: (i, 0)),
                   pl.BlockSpec((bm, fo), lambda i: (i, 0))],
        out_shape=[jax.ShapeDtypeStruct((n, fg), jnp.float32),
                   jax.ShapeDtypeStruct((n, fl), jnp.float32),
                   jax.ShapeDtypeStruct((n, fo), jnp.float32)],
    )(pre, stats, gcat, bcat, l1w)


def _tc_down_proj(d0, xg1, g1w, bm):
    m, k = d0.shape
    f = xg1.shape[1]
    fo = g1w.shape[1]
    steps = (m + bm - 1) // bm

    def body(d_ref, x_ref, w_ref, o_ref):
        t = jnp.dot(d_ref[:], x_ref[:], preferred_element_type=jnp.float32)
        o_ref[:] = jnp.dot(t, w_ref[:], preferred_element_type=jnp.float32)

    return pl.pallas_call(
        body,
        grid=(steps,),
        in_specs=[pl.BlockSpec((bm, k), lambda j: (j, 0)),
                  pl.BlockSpec((k, f), lambda j: (0, 0)),
                  pl.BlockSpec((f, fo), lambda j: (0, 0))],
        out_specs=pl.BlockSpec((bm, fo), lambda j: (j, 0)),
        out_shape=jax.ShapeDtypeStruct((m, fo), jnp.float32),
    )(d0, xg1, g1w)


def _tc_g2_head(parts, cntp, yg1, g1b, bg, bb, d1):
    n, f = yg1.shape
    m = d1.shape[0]

    def body(p_ref, c_ref, y_ref, b_ref, g_ref, b2_ref, d_ref, o_ref):
        cnt = jnp.sum(c_ref[:], axis=1, keepdims=True) + 1.0
        pre = (p_ref[0] + p_ref[1] + y_ref[:]) / cnt + b_ref[:]
        mean = jnp.mean(pre, axis=0, keepdims=True)
        var = jnp.mean(pre * pre, axis=0, keepdims=True) - mean * mean
        a = _leaky((pre - mean) * lax.rsqrt(var + _EPS) * g_ref[:] + b2_ref[:])
        xg2 = jnp.dot(d_ref[:], a, preferred_element_type=jnp.float32)
        o_ref[:] = _leaky(jnp.mean(xg2, axis=0, keepdims=True))

    return pl.pallas_call(
        body,
        grid=(1,),
        in_specs=[pl.BlockSpec((2, n, f), lambda i: (0, 0, 0)),
                  pl.BlockSpec((n, 32), lambda i: (0, 0)),
                  pl.BlockSpec((n, f), lambda i: (0, 0)),
                  pl.BlockSpec((1, f), lambda i: (0, 0)),
                  pl.BlockSpec((1, f), lambda i: (0, 0)),
                  pl.BlockSpec((1, f), lambda i: (0, 0)),
                  pl.BlockSpec((m, n), lambda i: (0, 0))],
        out_specs=pl.BlockSpec((1, f), lambda i: (0, 0)),
        out_shape=jax.ShapeDtypeStruct((1, f), jnp.float32),
    )(parts, cntp, yg1, g1b, bg, bb, d1)


def _tc_bn_apply(pre, stats, g, b, bm):
    n, f = pre.shape

    def body(pre_ref, st_ref, g_ref, b_ref, o_ref):
        mean, inv = _bn_from_stats(st_ref, float(n))
        o_ref[:] = _leaky((pre_ref[:] - mean) * inv * g_ref[:] + b_ref[:])

    return pl.pallas_call(
        body,
        grid=(n // bm,),
        in_specs=[pl.BlockSpec((bm, f), lambda i: (i, 0)),
                  pl.BlockSpec((2, f), lambda i: (0, 0)),
                  pl.BlockSpec((1, f), lambda i: (0, 0)),
                  pl.BlockSpec((1, f), lambda i: (0, 0))],
        out_specs=pl.BlockSpec((bm, f), lambda i: (i, 0)),
        out_shape=jax.ShapeDtypeStruct((n, f), jnp.float32),
    )(pre, stats, g, b)


def _tc_loc_final(xv, wloc, bloc, pooled, wenc, benc, bk):
    z, k = wloc.shape
    steps = k // bk

    def body(x_ref, w_ref, bl_ref, p_ref, we_ref, be_ref, o_ref, acc):
        j = pl.program_id(0)

        @pl.when(j == 0)
        def _():
            acc[:] = jnp.zeros_like(acc)

        acc[:] += lax.dot_general(x_ref[0], w_ref[:], (((1,), (1,)), ((), ())),
                                  preferred_element_type=jnp.float32)

        @pl.when(j == steps - 1)
        def _():
            loc = _leaky(acc[:] + bl_ref[:])
            zc = jnp.concatenate([p_ref[:], loc], axis=1)
            o_ref[:] = lax.dot_general(zc, we_ref[:], (((1,), (1,)), ((), ())),
                                       preferred_element_type=jnp.float32) + be_ref[:]

    return pl.pallas_call(
        body,
        grid=(steps,),
        in_specs=[pl.BlockSpec((1, 1, bk), lambda j: (j, 0, 0)),
                  pl.BlockSpec((z, bk), lambda j: (0, j)),
                  pl.BlockSpec((1, z), lambda j: (0, 0)),
                  pl.BlockSpec((1, pooled.shape[1]), lambda j: (0, 0)),
                  pl.BlockSpec(wenc.shape, lambda j: (0, 0)),
                  pl.BlockSpec((1, z), lambda j: (0, 0))],
        out_specs=pl.BlockSpec((1, z), lambda j: (0, 0)),
        out_shape=jax.ShapeDtypeStruct((1, z), jnp.float32),
        scratch_shapes=[pltpu.VMEM((1, z), jnp.float32)],
    )(xv, wloc, bloc, pooled, wenc, benc)


def kernel(x, batch_size, edge_index0, edge_index1, D0, D1,
           g0_W, g0_u, g0_c, g0_b, g1_W, g1_u, g1_c, g1_b,
           l0_W, l0_u, l0_c, l0_b, l1_W, l1_u, l1_c, l1_b,
           bng1_g, bng1_b, bng2_g, bng2_b, bnl1_g, bnl1_b, bnl2_g, bnl2_b,
           W_enc, b_enc, W_loc, b_loc):
    n0 = x.shape[0]
    n1 = D0.shape[0]
    fg1 = g0_W.shape[1]
    fl1 = l0_W.shape[1]
    fl2 = l1_W.shape[1]
    f01 = fg1 + fl1

    s = jnp.asarray(batch_size, jnp.float32)
    wcat = jnp.concatenate([g0_W, l0_W], axis=1) * s
    bcat1 = jnp.concatenate([g0_b, l0_b]).reshape(1, f01)
    gcat1 = jnp.concatenate([bng1_g, bnl1_g]).reshape(1, f01)
    gbcat1 = jnp.concatenate([bng1_b, bnl1_b]).reshape(1, f01)

    src0, dst0 = _prep_edges(edge_index0, n0, 40, 10240)
    src1, dst1 = _prep_edges(edge_index1, n1, 10, 4096)

    y0 = _tc_matmul(x, wcat, 1000)
    parts0, cnt0 = _sc_aggregate(y0, src0, dst0, 10240, 10240, True)
    cnt0t = cnt0.T
    pre1, st1 = _tc_combine_stats(parts0, cnt0t, y0, bcat1, 1000)
    xg1, xl1, y_l1 = _tc_bn_split(pre1, st1, gcat1, gbcat1, l1_W, fg1, 1000)

    partsl = _sc_aggregate(y_l1, src0, dst0, 10240, 10240, False)[0]

    y_g1 = _tc_down_proj(D0, xg1, g1_W, 128)
    partsg, cnt1 = _sc_aggregate(y_g1, src1, dst1, 4096, 2560, True)
    pooled = _tc_g2_head(partsg[:, :n1], cnt1.T[:n1], y_g1,
                         g1_b.reshape(1, -1), bng2_g.reshape(1, -1),
                         bng2_b.reshape(1, -1), D1)

    pre2, st2 = _tc_combine_stats(partsl, cnt0t, y_l1,
                                  l1_b.reshape(1, fl2), 1000)
    xl2 = _tc_bn_apply(pre2, st2, bnl2_g.reshape(1, fl2),
                       bnl2_b.reshape(1, fl2), 1000)

    return _tc_loc_final(xl2.reshape(10, 1, -1), W_loc, b_loc.reshape(1, -1),
                         pooled, W_enc, b_enc.reshape(1, -1), 16000)


# speedup vs baseline: 1.4264x; 1.4264x over previous
import functools

import jax
import jax.numpy as jnp
from jax import lax
from jax.experimental import pallas as pl
from jax.experimental.pallas import tpu as pltpu
from jax.experimental.pallas import tpu_sc as plsc

_SLOPE = 0.01
_EPS = 1e-5


def _leaky(x):
    return jnp.where(x > 0, x, _SLOPE * x)


def _fill_vmem(ref, rows, cols, val):
    v = jnp.full((16,), val, jnp.float32)

    def outer(r, _):
        def inner(cc, __):
            ref[r, pl.ds(cc * 16, 16)] = v
            return 0
        return lax.fori_loop(0, cols // 16, inner, 0)

    lax.fori_loop(0, rows, outer, 0)


def _sc_aggregate(y, src2d, dst2d, n_ch, n_nodes, n_acc, n_out, with_cnt, f0):
    F = y.shape[1]
    NB = 2
    N0 = int(round(n_ch * f0))
    N1 = n_ch - N0
    q0, r0 = divmod(N0, 16)
    q1, r1 = divmod(N1, 16)
    maxc = max(q0 + (1 if r0 else 0), q1 + (1 if r1 else 0))
    assert src2d.shape[0] >= n_ch + maxc and min(q0, q1) >= NB
    spare = n_acc - n_nodes - 1
    mesh = plsc.VectorSubcoreMesh(core_axis_name="c", subcore_axis_name="s")
    zr = n_acc // 16
    rpt = n_out // 16

    out_type = [jax.ShapeDtypeStruct((2, n_out, F), jnp.float32)]
    scratch = [
        pltpu.VMEM((maxc, 128), jnp.int32),
        pltpu.VMEM((maxc, 128), jnp.int32),
        pltpu.VMEM_SHARED((n_acc, F), jnp.float32),
    ]
    scratch += [pltpu.VMEM((128, F), jnp.float32) for _ in range(NB)]
    scratch += [pltpu.SemaphoreType.DMA for _ in range(NB)]
    scratch += [pltpu.SemaphoreType.DMA]
    if with_cnt:
        out_type.append(jax.ShapeDtypeStruct((32, n_acc), jnp.float32))
        scratch.append(pltpu.VMEM((n_acc,), jnp.float32))

    def body(y_hbm, src_hbm, dst_hbm, out_hbm, *rest):
        if with_cnt:
            cnt_hbm, rest = rest[0], rest[1:]
        src_v, dst_v, acc_sh = rest[:3]
        rows = rest[3:3 + NB]
        gsems = rest[3 + NB:3 + 2 * NB]
        ssem = rest[3 + 2 * NB]
        cnt_v = rest[4 + 2 * NB] if with_cnt else None
        cid = lax.axis_index("c")
        sid = lax.axis_index("s")
        wid = cid * 16 + sid
        ones16 = jnp.full((16,), 1.0, jnp.float32)
        zero16 = jnp.zeros((16,), jnp.float32)
        is0 = cid == 0
        count = jnp.where(is0, q0 + (sid < r0).astype(jnp.int32),
                          q1 + (sid < r1).astype(jnp.int32))
        start = jnp.where(is0, sid * q0 + jnp.minimum(sid, r0),
                          N0 + sid * q1 + jnp.minimum(sid, r1))

        _fill_vmem(rows[0], 128, F, 0.0)

        def zacc(k, _):
            pltpu.sync_copy(rows[0], acc_sh.at[pl.ds(sid * zr + k * 128, 128)])
            return 0
        lax.fori_loop(0, zr // 128, zacc, 0)
        if with_cnt:
            def zcnt(i, _):
                cnt_v[pl.ds(i * 16, 16)] = zero16
                return 0
            lax.fori_loop(0, n_acc // 16, zcnt, 0)

        pltpu.sync_copy(src_hbm.at[pl.ds(start, maxc)], src_v)
        pltpu.sync_copy(dst_hbm.at[pl.ds(start, maxc)], dst_v)

        def sel(j, _):
            dummy = n_nodes + lax.rem(start + j, spare)
            for c in range(8):
                s16 = src_v[j, pl.ds(c * 16, 16)]
                d16 = dst_v[j, pl.ds(c * 16, 16)]
                dst_v[j, pl.ds(c * 16, 16)] = jnp.where(
                    s16 == d16, jnp.zeros((16,), jnp.int32) + dummy, d16)
            return 0
        lax.fori_loop(0, count, sel, 0)
        plsc.subcore_barrier()

        for b in range(NB):
            pltpu.async_copy(y_hbm.at[src_v.at[b]], rows[b], gsems[b])

        def _chunk(j, b):
            pltpu.make_async_copy(y_hbm.at[src_v.at[j]], rows[b],
                                  gsems[b]).wait()
            d = pltpu.async_copy(rows[b], acc_sh.at[dst_v.at[j]], ssem,
                                 add=True)
            if with_cnt:
                for c in range(8):
                    idx = dst_v[j, pl.ds(c * 16, 16)]
                    plsc.addupdate_scatter(cnt_v, [idx], ones16)
            d.wait()

        nj = count // NB

        def step(jj, _):
            for b in range(NB):
                j = jj * NB + b
                _chunk(j, b)

                @pl.when(j + NB < count)
                def _():
                    pltpu.async_copy(y_hbm.at[src_v.at[j + NB]], rows[b],
                                     gsems[b])
            return 0
        lax.fori_loop(0, nj, step, 0)
        for b in range(NB):
            t = nj * NB + b

            @pl.when(t < count)
            def _():
                _chunk(t, b)

        plsc.subcore_barrier()
        pltpu.sync_copy(acc_sh.at[pl.ds(sid * rpt, rpt)],
                        out_hbm.at[cid, pl.ds(sid * rpt, rpt)])
        if with_cnt:
            pltpu.sync_copy(cnt_v, cnt_hbm.at[wid])

    fn = pl.kernel(body, out_type=out_type, mesh=mesh, scratch_types=scratch,
                   compiler_params=pltpu.CompilerParams(
                       use_tc_tiling_on_sc=False, needs_layout_passes=False))
    return fn(y, src2d, dst2d)


def _prep_edges(ei, n_ch, pad_rows):
    tot = (n_ch + pad_rows) * 128
    e = ei.shape[1]
    z = jnp.zeros((tot - e,), jnp.int32)
    return (jnp.concatenate([ei[0], z]).reshape(-1, 128),
            jnp.concatenate([ei[1], z]).reshape(-1, 128))


def _tc_matmul(x, w, bm):
    n, k = x.shape
    f = w.shape[1]

    def body(x_ref, w_ref, o_ref):
        o_ref[:] = jnp.dot(x_ref[:], w_ref[:], preferred_element_type=jnp.float32)

    return pl.pallas_call(
        body,
        grid=(n // bm,),
        in_specs=[pl.BlockSpec((bm, k), lambda i: (i, 0)),
                  pl.BlockSpec((k, f), lambda i: (0, 0))],
        out_specs=pl.BlockSpec((bm, f), lambda i: (i, 0)),
        out_shape=jax.ShapeDtypeStruct((n, f), jnp.float32),
    )(x, w)


def _tc_combine_stats(parts, cntp, y, b, bm):
    n, f = y.shape
    steps = n // bm

    def body(p_ref, c_ref, y_ref, b_ref, pre_ref, st_ref, sacc):
        i = pl.program_id(0)

        @pl.when(i == 0)
        def _():
            sacc[:] = jnp.zeros_like(sacc)

        cnt = jnp.sum(c_ref[:], axis=1, keepdims=True) + 1.0
        pre = (p_ref[0] + p_ref[1] + y_ref[:]) / cnt + b_ref[:]
        pre_ref[:] = pre
        sacc[0:1, :] += jnp.sum(pre, axis=0, keepdims=True)
        sacc[1:2, :] += jnp.sum(pre * pre, axis=0, keepdims=True)

        @pl.when(i == steps - 1)
        def _():
            st_ref[:] = sacc[:]

    return pl.pallas_call(
        body,
        grid=(steps,),
        in_specs=[pl.BlockSpec((2, bm, f), lambda i: (0, i, 0)),
                  pl.BlockSpec((bm, 32), lambda i: (i, 0)),
                  pl.BlockSpec((bm, f), lambda i: (i, 0)),
                  pl.BlockSpec((1, f), lambda i: (0, 0))],
        out_specs=[pl.BlockSpec((bm, f), lambda i: (i, 0)),
                   pl.BlockSpec((2, f), lambda i: (0, 0))],
        out_shape=[jax.ShapeDtypeStruct((n, f), jnp.float32),
                   jax.ShapeDtypeStruct((2, f), jnp.float32)],
        scratch_shapes=[pltpu.VMEM((2, f), jnp.float32)],
    )(parts, cntp, y, b)


def _bn_from_stats(st_ref, n):
    mean = st_ref[0:1, :] / n
    var = st_ref[1:2, :] / n - (st_ref[0:1, :] / n) ** 2
    inv = lax.rsqrt(var + _EPS)
    return mean, inv


def _tc_bn_split(pre, stats, gcat, bcat, l1w, fg, bm):
    n, f = pre.shape
    fl = f - fg
    fo = l1w.shape[1]

    def body(pre_ref, st_ref, g_ref, b_ref, w_ref, xg_ref, xl_ref, yl_ref):
        mean, inv = _bn_from_stats(st_ref, float(n))
        z = (pre_ref[:] - mean) * inv * g_ref[:] + b_ref[:]
        a = _leaky(z)
        xg_ref[:] = a[:, :fg]
        xl = a[:, fg:]
        xl_ref[:] = xl
        yl_ref[:] = jnp.dot(xl, w_ref[:], preferred_element_type=jnp.float32)

    return pl.pallas_call(
        body,
        grid=(n // bm,),
        in_specs=[pl.BlockSpec((bm, f), lambda i: (i, 0)),
                  pl.BlockSpec((2, f), lambda i: (0, 0)),
                  pl.BlockSpec((1, f), lambda i: (0, 0)),
                  pl.BlockSpec((1, f), lambda i: (0, 0)),
                  pl.BlockSpec((fl, fo), lambda i: (0, 0))],
        out_specs=[pl.BlockSpec((bm, fg), lambda i: (i, 0)),
                   pl.BlockSpec((bm, fl), lambda i: (i, 0)),
                   pl.BlockSpec((bm, fo), lambda i: (i, 0))],
        out_shape=[jax.ShapeDtypeStruct((n, fg), jnp.float32),
                   jax.ShapeDtypeStruct((n, fl), jnp.float32),
                   jax.ShapeDtypeStruct((n, fo), jnp.float32)],
    )(pre, stats, gcat, bcat, l1w)


def _tc_down_proj(d0, xg1, g1w, bm):
    m, k = d0.shape
    f = xg1.shape[1]
    fo = g1w.shape[1]
    steps = (m + bm - 1) // bm

    def body(d_ref, x_ref, w_ref, o_ref):
        t = jnp.dot(d_ref[:], x_ref[:], preferred_element_type=jnp.float32)
        o_ref[:] = jnp.dot(t, w_ref[:], preferred_element_type=jnp.float32)

    return pl.pallas_call(
        body,
        grid=(steps,),
        in_specs=[pl.BlockSpec((bm, k), lambda j: (j, 0)),
                  pl.BlockSpec((k, f), lambda j: (0, 0)),
                  pl.BlockSpec((f, fo), lambda j: (0, 0))],
        out_specs=pl.BlockSpec((bm, fo), lambda j: (j, 0)),
        out_shape=jax.ShapeDtypeStruct((m, fo), jnp.float32),
    )(d0, xg1, g1w)


def _tc_g2_head(parts, cntp, yg1, g1b, bg, bb, d1):
    n, f = yg1.shape
    m = d1.shape[0]

    def body(p_ref, c_ref, y_ref, b_ref, g_ref, b2_ref, d_ref, o_ref):
        cnt = jnp.sum(c_ref[:], axis=1, keepdims=True) + 1.0
        pre = (p_ref[0] + p_ref[1] + y_ref[:]) / cnt + b_ref[:]
        mean = jnp.mean(pre, axis=0, keepdims=True)
        var = jnp.mean(pre * pre, axis=0, keepdims=True) - mean * mean
        a = _leaky((pre - mean) * lax.rsqrt(var + _EPS) * g_ref[:] + b2_ref[:])
        xg2 = jnp.dot(d_ref[:], a, preferred_element_type=jnp.float32)
        o_ref[:] = _leaky(jnp.mean(xg2, axis=0, keepdims=True))

    return pl.pallas_call(
        body,
        grid=(1,),
        in_specs=[pl.BlockSpec((2, n, f), lambda i: (0, 0, 0)),
                  pl.BlockSpec((n, 32), lambda i: (0, 0)),
                  pl.BlockSpec((n, f), lambda i: (0, 0)),
                  pl.BlockSpec((1, f), lambda i: (0, 0)),
                  pl.BlockSpec((1, f), lambda i: (0, 0)),
                  pl.BlockSpec((1, f), lambda i: (0, 0)),
                  pl.BlockSpec((m, n), lambda i: (0, 0))],
        out_specs=pl.BlockSpec((1, f), lambda i: (0, 0)),
        out_shape=jax.ShapeDtypeStruct((1, f), jnp.float32),
    )(parts, cntp, yg1, g1b, bg, bb, d1)


def _tc_bn_apply(pre, stats, g, b, bm):
    n, f = pre.shape

    def body(pre_ref, st_ref, g_ref, b_ref, o_ref):
        mean, inv = _bn_from_stats(st_ref, float(n))
        o_ref[:] = _leaky((pre_ref[:] - mean) * inv * g_ref[:] + b_ref[:])

    return pl.pallas_call(
        body,
        grid=(n // bm,),
        in_specs=[pl.BlockSpec((bm, f), lambda i: (i, 0)),
                  pl.BlockSpec((2, f), lambda i: (0, 0)),
                  pl.BlockSpec((1, f), lambda i: (0, 0)),
                  pl.BlockSpec((1, f), lambda i: (0, 0))],
        out_specs=pl.BlockSpec((bm, f), lambda i: (i, 0)),
        out_shape=jax.ShapeDtypeStruct((n, f), jnp.float32),
    )(pre, stats, g, b)


def _tc_loc_final(xv, wloc, bloc, pooled, wenc, benc, bk):
    z, k = wloc.shape
    steps = k // bk

    def body(x_ref, w_ref, bl_ref, p_ref, we_ref, be_ref, o_ref, acc):
        j = pl.program_id(0)

        @pl.when(j == 0)
        def _():
            acc[:] = jnp.zeros_like(acc)

        acc[:] += lax.dot_general(x_ref[0], w_ref[:], (((1,), (1,)), ((), ())),
                                  preferred_element_type=jnp.float32)

        @pl.when(j == steps - 1)
        def _():
            loc = _leaky(acc[:] + bl_ref[:])
            zc = jnp.concatenate([p_ref[:], loc], axis=1)
            o_ref[:] = lax.dot_general(zc, we_ref[:], (((1,), (1,)), ((), ())),
                                       preferred_element_type=jnp.float32) + be_ref[:]

    return pl.pallas_call(
        body,
        grid=(steps,),
        in_specs=[pl.BlockSpec((1, 1, bk), lambda j: (j, 0, 0)),
                  pl.BlockSpec((z, bk), lambda j: (0, j)),
                  pl.BlockSpec((1, z), lambda j: (0, 0)),
                  pl.BlockSpec((1, pooled.shape[1]), lambda j: (0, 0)),
                  pl.BlockSpec(wenc.shape, lambda j: (0, 0)),
                  pl.BlockSpec((1, z), lambda j: (0, 0))],
        out_specs=pl.BlockSpec((1, z), lambda j: (0, 0)),
        out_shape=jax.ShapeDtypeStruct((1, z), jnp.float32),
        scratch_shapes=[pltpu.VMEM((1, z), jnp.float32)],
    )(xv, wloc, bloc, pooled, wenc, benc)


def kernel(x, batch_size, edge_index0, edge_index1, D0, D1,
           g0_W, g0_u, g0_c, g0_b, g1_W, g1_u, g1_c, g1_b,
           l0_W, l0_u, l0_c, l0_b, l1_W, l1_u, l1_c, l1_b,
           bng1_g, bng1_b, bng2_g, bng2_b, bnl1_g, bnl1_b, bnl2_g, bnl2_b,
           W_enc, b_enc, W_loc, b_loc):
    n0 = x.shape[0]
    n1 = D0.shape[0]
    fg1 = g0_W.shape[1]
    fl1 = l0_W.shape[1]
    fl2 = l1_W.shape[1]
    f01 = fg1 + fl1

    s = jnp.asarray(batch_size, jnp.float32)
    wcat = jnp.concatenate([g0_W, l0_W], axis=1) * s
    bcat1 = jnp.concatenate([g0_b, l0_b]).reshape(1, f01)
    gcat1 = jnp.concatenate([bng1_g, bnl1_g]).reshape(1, f01)
    gbcat1 = jnp.concatenate([bng1_b, bnl1_b]).reshape(1, f01)

    src0, dst0 = _prep_edges(edge_index0, 1250, 64)
    src1, dst1 = _prep_edges(edge_index1, 320, 64)

    y0 = _tc_matmul(x, wcat, 1000)
    parts0, cnt0 = _sc_aggregate(y0, src0, dst0, 1250, n0, 10240, 10240,
                                 True, 0.72)
    cnt0t = cnt0.T
    pre1, st1 = _tc_combine_stats(parts0, cnt0t, y0, bcat1, 1000)
    xg1, xl1, y_l1 = _tc_bn_split(pre1, st1, gcat1, gbcat1, l1_W, fg1, 1000)

    partsl = _sc_aggregate(y_l1, src0, dst0, 1250, n0, 10240, 10240,
                           False, 0.57)[0]

    y_g1 = _tc_down_proj(D0, xg1, g1_W, 128)
    partsg, cnt1 = _sc_aggregate(y_g1, src1, dst1, 320, n1, 4096, 2560,
                                 True, 0.55)
    pooled = _tc_g2_head(partsg[:, :n1], cnt1.T[:n1], y_g1,
                         g1_b.reshape(1, -1), bng2_g.reshape(1, -1),
                         bng2_b.reshape(1, -1), D1)

    pre2, st2 = _tc_combine_stats(partsl, cnt0t, y_l1,
                                  l1_b.reshape(1, fl2), 1000)
    xl2 = _tc_bn_apply(pre2, st2, bnl2_g.reshape(1, fl2),
                       bnl2_b.reshape(1, fl2), 1000)

    return _tc_loc_final(xl2.reshape(10, 1, -1), W_loc, b_loc.reshape(1, -1),
                         pooled, W_enc, b_enc.reshape(1, -1), 16000)


# speedup vs baseline: 1.4358x; 1.0066x over previous
import functools

import jax
import jax.numpy as jnp
from jax import lax
from jax.experimental import pallas as pl
from jax.experimental.pallas import tpu as pltpu
from jax.experimental.pallas import tpu_sc as plsc

_SLOPE = 0.01
_EPS = 1e-5


def _leaky(x):
    return jnp.where(x > 0, x, _SLOPE * x)


def _fill_vmem(ref, rows, cols, val):
    v = jnp.full((16,), val, jnp.float32)

    def outer(r, _):
        def inner(cc, __):
            ref[r, pl.ds(cc * 16, 16)] = v
            return 0
        return lax.fori_loop(0, cols // 16, inner, 0)

    lax.fori_loop(0, rows, outer, 0)


def _sc_aggregate(y, src2d, dst2d, n_ch, n_nodes, n_acc, n_out, f0):
    F = y.shape[1]
    NB = 2
    N0 = int(round(n_ch * f0))
    N1 = n_ch - N0
    q0, r0 = divmod(N0, 16)
    q1, r1 = divmod(N1, 16)
    maxc = max(q0 + (1 if r0 else 0), q1 + (1 if r1 else 0))
    assert src2d.shape[0] >= n_ch and min(q0, q1) >= NB
    spare = n_acc - n_nodes - 1
    mesh = plsc.VectorSubcoreMesh(core_axis_name="c", subcore_axis_name="s")
    zr = n_acc // 16
    rpt = n_out // 16

    out_type = [jax.ShapeDtypeStruct((2, n_out, F), jnp.float32)]
    scratch = [
        pltpu.VMEM((maxc, 128), jnp.int32),
        pltpu.VMEM((maxc, 128), jnp.int32),
        pltpu.VMEM_SHARED((n_acc, F), jnp.float32),
    ]
    scratch += [pltpu.VMEM((128, F), jnp.float32) for _ in range(NB)]
    scratch += [pltpu.SemaphoreType.DMA for _ in range(NB)]
    scratch += [pltpu.SemaphoreType.DMA]

    def body(y_hbm, src_hbm, dst_hbm, out_hbm, *rest):
        src_v, dst_v, acc_sh = rest[:3]
        rows = rest[3:3 + NB]
        gsems = rest[3 + NB:3 + 2 * NB]
        ssem = rest[3 + 2 * NB]
        cid = lax.axis_index("c")
        sid = lax.axis_index("s")
        is0 = cid == 0
        count = jnp.where(is0, q0 + (sid < r0).astype(jnp.int32),
                          q1 + (sid < r1).astype(jnp.int32))
        start = jnp.where(is0, sid * q0 + jnp.minimum(sid, r0),
                          N0 + sid * q1 + jnp.minimum(sid, r1))
        start_l = jnp.minimum(start, n_ch - maxc)
        off = start - start_l

        _fill_vmem(rows[0], 128, F, 0.0)

        def zacc(k, _):
            pltpu.sync_copy(rows[0], acc_sh.at[pl.ds(sid * zr + k * 128, 128)])
            return 0
        lax.fori_loop(0, zr // 128, zacc, 0)

        pltpu.sync_copy(src_hbm.at[pl.ds(start_l, maxc)], src_v)
        pltpu.sync_copy(dst_hbm.at[pl.ds(start_l, maxc)], dst_v)

        def sel(j, _):
            dummy = n_nodes + lax.rem(start + j, spare)
            for c in range(8):
                s16 = src_v[off + j, pl.ds(c * 16, 16)]
                d16 = dst_v[off + j, pl.ds(c * 16, 16)]
                dst_v[off + j, pl.ds(c * 16, 16)] = jnp.where(
                    s16 == d16, jnp.zeros((16,), jnp.int32) + dummy, d16)
            return 0
        lax.fori_loop(0, count, sel, 0)
        plsc.subcore_barrier()

        for b in range(NB):
            pltpu.async_copy(y_hbm.at[src_v.at[off + b]], rows[b], gsems[b])

        def _chunk(j, b):
            pltpu.make_async_copy(y_hbm.at[src_v.at[off + j]], rows[b],
                                  gsems[b]).wait()
            pltpu.async_copy(rows[b], acc_sh.at[dst_v.at[off + j]], ssem,
                             add=True).wait()

        nj = count // NB

        def step(jj, _):
            for b in range(NB):
                j = jj * NB + b
                _chunk(j, b)

                @pl.when(j + NB < count)
                def _():
                    pltpu.async_copy(y_hbm.at[src_v.at[off + j + NB]],
                                     rows[b], gsems[b])
            return 0
        lax.fori_loop(0, nj, step, 0)
        for b in range(NB):
            t = nj * NB + b

            @pl.when(t < count)
            def _():
                _chunk(t, b)

        plsc.subcore_barrier()
        pltpu.sync_copy(acc_sh.at[pl.ds(sid * rpt, rpt)],
                        out_hbm.at[cid, pl.ds(sid * rpt, rpt)])

    fn = pl.kernel(body, out_type=out_type, mesh=mesh, scratch_types=scratch,
                   compiler_params=pltpu.CompilerParams(
                       use_tc_tiling_on_sc=False, needs_layout_passes=False))
    return fn(y, src2d, dst2d)[0]


def _prep_edges(ei, n_ch):
    e = ei.shape[1]
    if e < n_ch * 128:
        z = jnp.zeros((n_ch * 128 - e,), jnp.int32)
        return (jnp.concatenate([ei[0], z]).reshape(-1, 128),
                jnp.concatenate([ei[1], z]).reshape(-1, 128))
    return ei[0].reshape(-1, 128), ei[1].reshape(-1, 128)


def _tc_matmul(x, w, bm, ones_cols=0):
    n, k = x.shape
    f = w.shape[1]
    fo = f + ones_cols

    def body(x_ref, w_ref, o_ref):
        o_ref[:, :f] = jnp.dot(x_ref[:], w_ref[:],
                               preferred_element_type=jnp.float32)
        if ones_cols:
            o_ref[:, f:] = jnp.ones((bm, ones_cols), jnp.float32)

    return pl.pallas_call(
        body,
        grid=(n // bm,),
        in_specs=[pl.BlockSpec((bm, k), lambda i: (i, 0)),
                  pl.BlockSpec((k, f), lambda i: (0, 0))],
        out_specs=pl.BlockSpec((bm, fo), lambda i: (i, 0)),
        out_shape=jax.ShapeDtypeStruct((n, fo), jnp.float32),
    )(x, w)


def _tc_combine_stats1(parts, y, b, fdat, bm):
    n, fall = y.shape
    steps = n // bm

    def body(p_ref, y_ref, b_ref, pre_ref, st_ref, cnt_ref, sacc):
        i = pl.program_id(0)

        @pl.when(i == 0)
        def _():
            sacc[:] = jnp.zeros_like(sacc)

        raw = p_ref[0] + p_ref[1] + y_ref[:]
        cnt = raw[:, fdat:fdat + 1]
        pre = raw[:, :fdat] / cnt + b_ref[:]
        pre_ref[:] = pre
        cnt_ref[:] = cnt
        sacc[0:1, :] += jnp.sum(pre, axis=0, keepdims=True)
        sacc[1:2, :] += jnp.sum(pre * pre, axis=0, keepdims=True)

        @pl.when(i == steps - 1)
        def _():
            st_ref[:] = sacc[:]

    return pl.pallas_call(
        body,
        grid=(steps,),
        in_specs=[pl.BlockSpec((2, bm, fall), lambda i: (0, i, 0)),
                  pl.BlockSpec((bm, fall), lambda i: (i, 0)),
                  pl.BlockSpec((1, fdat), lambda i: (0, 0))],
        out_specs=[pl.BlockSpec((bm, fdat), lambda i: (i, 0)),
                   pl.BlockSpec((2, fdat), lambda i: (0, 0)),
                   pl.BlockSpec((bm, 1), lambda i: (i, 0))],
        out_shape=[jax.ShapeDtypeStruct((n, fdat), jnp.float32),
                   jax.ShapeDtypeStruct((2, fdat), jnp.float32),
                   jax.ShapeDtypeStruct((n, 1), jnp.float32)],
        scratch_shapes=[pltpu.VMEM((2, fdat), jnp.float32)],
    )(parts, y, b)


def _tc_combine_stats2(parts, y, cnt, b, bm):
    n, f = y.shape
    steps = n // bm

    def body(p_ref, y_ref, c_ref, b_ref, pre_ref, st_ref, sacc):
        i = pl.program_id(0)

        @pl.when(i == 0)
        def _():
            sacc[:] = jnp.zeros_like(sacc)

        pre = (p_ref[0] + p_ref[1] + y_ref[:]) / c_ref[:] + b_ref[:]
        pre_ref[:] = pre
        sacc[0:1, :] += jnp.sum(pre, axis=0, keepdims=True)
        sacc[1:2, :] += jnp.sum(pre * pre, axis=0, keepdims=True)

        @pl.when(i == steps - 1)
        def _():
            st_ref[:] = sacc[:]

    return pl.pallas_call(
        body,
        grid=(steps,),
        in_specs=[pl.BlockSpec((2, bm, f), lambda i: (0, i, 0)),
                  pl.BlockSpec((bm, f), lambda i: (i, 0)),
                  pl.BlockSpec((bm, 1), lambda i: (i, 0)),
                  pl.BlockSpec((1, f), lambda i: (0, 0))],
        out_specs=[pl.BlockSpec((bm, f), lambda i: (i, 0)),
                   pl.BlockSpec((2, f), lambda i: (0, 0))],
        out_shape=[jax.ShapeDtypeStruct((n, f), jnp.float32),
                   jax.ShapeDtypeStruct((2, f), jnp.float32)],
        scratch_shapes=[pltpu.VMEM((2, f), jnp.float32)],
    )(parts, y, cnt, b)


def _bn_from_stats(st_ref, n):
    mean = st_ref[0:1, :] / n
    var = st_ref[1:2, :] / n - (st_ref[0:1, :] / n) ** 2
    inv = lax.rsqrt(var + _EPS)
    return mean, inv


def _tc_bn_split(pre, stats, gcat, bcat, l1w, fg, bm):
    n, f = pre.shape
    fl = f - fg
    fo = l1w.shape[1]

    def body(pre_ref, st_ref, g_ref, b_ref, w_ref, xg_ref, xl_ref, yl_ref):
        mean, inv = _bn_from_stats(st_ref, float(n))
        z = (pre_ref[:] - mean) * inv * g_ref[:] + b_ref[:]
        a = _leaky(z)
        xg_ref[:] = a[:, :fg]
        xl = a[:, fg:]
        xl_ref[:] = xl
        yl_ref[:] = jnp.dot(xl, w_ref[:], preferred_element_type=jnp.float32)

    return pl.pallas_call(
        body,
        grid=(n // bm,),
        in_specs=[pl.BlockSpec((bm, f), lambda i: (i, 0)),
                  pl.BlockSpec((2, f), lambda i: (0, 0)),
                  pl.BlockSpec((1, f), lambda i: (0, 0)),
                  pl.BlockSpec((1, f), lambda i: (0, 0)),
                  pl.BlockSpec((fl, fo), lambda i: (0, 0))],
        out_specs=[pl.BlockSpec((bm, fg), lambda i: (i, 0)),
                   pl.BlockSpec((bm, fl), lambda i: (i, 0)),
                   pl.BlockSpec((bm, fo), lambda i: (i, 0))],
        out_shape=[jax.ShapeDtypeStruct((n, fg), jnp.float32),
                   jax.ShapeDtypeStruct((n, fl), jnp.float32),
                   jax.ShapeDtypeStruct((n, fo), jnp.float32)],
    )(pre, stats, gcat, bcat, l1w)


def _tc_down_proj(d0, xg1, g1w, bm, ones_cols):
    m, k = d0.shape
    f = xg1.shape[1]
    fo = g1w.shape[1]
    fa = fo + ones_cols
    steps = (m + bm - 1) // bm

    def body(d_ref, x_ref, w_ref, o_ref):
        t = jnp.dot(d_ref[:], x_ref[:], preferred_element_type=jnp.float32)
        o_ref[:, :fo] = jnp.dot(t, w_ref[:], preferred_element_type=jnp.float32)
        if ones_cols:
            o_ref[:, fo:] = jnp.ones((bm, ones_cols), jnp.float32)

    return pl.pallas_call(
        body,
        grid=(steps,),
        in_specs=[pl.BlockSpec((bm, k), lambda j: (j, 0)),
                  pl.BlockSpec((k, f), lambda j: (0, 0)),
                  pl.BlockSpec((f, fo), lambda j: (0, 0))],
        out_specs=pl.BlockSpec((bm, fa), lambda j: (j, 0)),
        out_shape=jax.ShapeDtypeStruct((m, fa), jnp.float32),
    )(d0, xg1, g1w)


def _tc_g2_head(parts, yg1, g1b, bg, bb, d1, fdat):
    n, fall = yg1.shape
    m = d1.shape[0]

    def body(p_ref, y_ref, b_ref, g_ref, b2_ref, d_ref, o_ref):
        raw = p_ref[0] + p_ref[1] + y_ref[:]
        cnt = raw[:, fdat:fdat + 1]
        pre = raw[:, :fdat] / cnt + b_ref[:]
        mean = jnp.mean(pre, axis=0, keepdims=True)
        var = jnp.mean(pre * pre, axis=0, keepdims=True) - mean * mean
        a = _leaky((pre - mean) * lax.rsqrt(var + _EPS) * g_ref[:] + b2_ref[:])
        xg2 = jnp.dot(d_ref[:], a, preferred_element_type=jnp.float32)
        o_ref[:] = _leaky(jnp.mean(xg2, axis=0, keepdims=True))

    return pl.pallas_call(
        body,
        grid=(1,),
        in_specs=[pl.BlockSpec((2, n, fall), lambda i: (0, 0, 0)),
                  pl.BlockSpec((n, fall), lambda i: (0, 0)),
                  pl.BlockSpec((1, fdat), lambda i: (0, 0)),
                  pl.BlockSpec((1, fdat), lambda i: (0, 0)),
                  pl.BlockSpec((1, fdat), lambda i: (0, 0)),
                  pl.BlockSpec((m, n), lambda i: (0, 0))],
        out_specs=pl.BlockSpec((1, fdat), lambda i: (0, 0)),
        out_shape=jax.ShapeDtypeStruct((1, fdat), jnp.float32),
    )(parts, yg1, g1b, bg, bb, d1)


def _tc_bn_apply(pre, stats, g, b, bm):
    n, f = pre.shape

    def body(pre_ref, st_ref, g_ref, b_ref, o_ref):
        mean, inv = _bn_from_stats(st_ref, float(n))
        o_ref[:] = _leaky((pre_ref[:] - mean) * inv * g_ref[:] + b_ref[:])

    return pl.pallas_call(
        body,
        grid=(n // bm,),
        in_specs=[pl.BlockSpec((bm, f), lambda i: (i, 0)),
                  pl.BlockSpec((2, f), lambda i: (0, 0)),
                  pl.BlockSpec((1, f), lambda i: (0, 0)),
                  pl.BlockSpec((1, f), lambda i: (0, 0))],
        out_specs=pl.BlockSpec((bm, f), lambda i: (i, 0)),
        out_shape=jax.ShapeDtypeStruct((n, f), jnp.float32),
    )(pre, stats, g, b)


def _tc_loc_final(xv, wloc, bloc, pooled, wenc, benc, bk):
    z, k = wloc.shape
    steps = k // bk

    def body(x_ref, w_ref, bl_ref, p_ref, we_ref, be_ref, o_ref, acc):
        j = pl.program_id(0)

        @pl.when(j == 0)
        def _():
            acc[:] = jnp.zeros_like(acc)

        acc[:] += lax.dot_general(x_ref[0], w_ref[:], (((1,), (1,)), ((), ())),
                                  preferred_element_type=jnp.float32)

        @pl.when(j == steps - 1)
        def _():
            loc = _leaky(acc[:] + bl_ref[:])
            zc = jnp.concatenate([p_ref[:], loc], axis=1)
            o_ref[:] = lax.dot_general(zc, we_ref[:], (((1,), (1,)), ((), ())),
                                       preferred_element_type=jnp.float32) + be_ref[:]

    return pl.pallas_call(
        body,
        grid=(steps,),
        in_specs=[pl.BlockSpec((1, 1, bk), lambda j: (j, 0, 0)),
                  pl.BlockSpec((z, bk), lambda j: (0, j)),
                  pl.BlockSpec((1, z), lambda j: (0, 0)),
                  pl.BlockSpec((1, pooled.shape[1]), lambda j: (0, 0)),
                  pl.BlockSpec(wenc.shape, lambda j: (0, 0)),
                  pl.BlockSpec((1, z), lambda j: (0, 0))],
        out_specs=pl.BlockSpec((1, z), lambda j: (0, 0)),
        out_shape=jax.ShapeDtypeStruct((1, z), jnp.float32),
        scratch_shapes=[pltpu.VMEM((1, z), jnp.float32)],
    )(xv, wloc, bloc, pooled, wenc, benc)


def kernel(x, batch_size, edge_index0, edge_index1, D0, D1,
           g0_W, g0_u, g0_c, g0_b, g1_W, g1_u, g1_c, g1_b,
           l0_W, l0_u, l0_c, l0_b, l1_W, l1_u, l1_c, l1_b,
           bng1_g, bng1_b, bng2_g, bng2_b, bnl1_g, bnl1_b, bnl2_g, bnl2_b,
           W_enc, b_enc, W_loc, b_loc):
    n0 = x.shape[0]
    n1 = D0.shape[0]
    fg1 = g0_W.shape[1]
    fl1 = l0_W.shape[1]
    fl2 = l1_W.shape[1]
    f01 = fg1 + fl1

    s = jnp.asarray(batch_size, jnp.float32)
    wcat = jnp.concatenate([g0_W, l0_W], axis=1) * s
    bcat1 = jnp.concatenate([g0_b, l0_b]).reshape(1, f01)
    gcat1 = jnp.concatenate([bng1_g, bnl1_g]).reshape(1, f01)
    gbcat1 = jnp.concatenate([bng1_b, bnl1_b]).reshape(1, f01)

    src0, dst0 = _prep_edges(edge_index0, 1250)
    src1, dst1 = _prep_edges(edge_index1, 320)

    y0 = _tc_matmul(x, wcat, 1000, 16)
    parts0 = _sc_aggregate(y0, src0, dst0, 1250, n0, 10240, 10240, 0.60)
    pre1, st1, cnt0 = _tc_combine_stats1(parts0, y0, bcat1, f01, 1000)
    xg1, xl1, y_l1 = _tc_bn_split(pre1, st1, gcat1, gbcat1, l1_W, fg1, 1000)

    partsl = _sc_aggregate(y_l1, src0, dst0, 1250, n0, 10240, 10240, 0.62)

    y_g1 = _tc_down_proj(D0, xg1, g1_W, 128, 16)
    partsg = _sc_aggregate(y_g1, src1, dst1, 320, n1, 4096, 2560, 0.50)

    pre2, st2 = _tc_combine_stats2(partsl, y_l1, cnt0,
                                   l1_b.reshape(1, fl2), 1000)
    xl2 = _tc_bn_apply(pre2, st2, bnl2_g.reshape(1, fl2),
                       bnl2_b.reshape(1, fl2), 1000)

    pooled = _tc_g2_head(partsg[:, :n1], y_g1, g1_b.reshape(1, -1),
                         bng2_g.reshape(1, -1), bng2_b.reshape(1, -1),
                         D1, 32)
    return _tc_loc_final(xl2.reshape(10, 1, -1), W_loc, b_loc.reshape(1, -1),
                         pooled, W_enc, b_enc.reshape(1, -1), 16000)
